# Initial kernel scaffold; baseline (speedup 1.0000x reference)
import jax
import jax.numpy as jnp
from jax.experimental import pallas as pl


def kernel(atom_z, nbr_fea, nbr_idx, crystal_atom_idx, atom_emb, W_full_0, b_full_0, bn_g_0, bn_b_0, W_full_1, b_full_1, bn_g_1, bn_b_1, W_full_2, b_full_2, bn_g_2, bn_b_2, W1, b1, W2, b2, Wo, bo):
    raise NotImplementedError("write your pallas kernel here")


# speedup vs baseline: 1.0775x; 1.0775x over previous
import functools

import jax
import jax.numpy as jnp
from jax import lax
from jax.experimental import pallas as pl
from jax.experimental.pallas import tpu as pltpu
from jax.experimental.pallas import tpu_sc as plsc

N = 50000
M = 16
F = 64
NBR = 16
NCRY = 256
H = 128
EPS = 1e-5

B = 400
NBLK = N // B
E_TOT = N * M

LANES = 128
NW = 32

NBR_IDX_ROWS = 6400
NBR_ROWS_PER_W = NBR_IDX_ROWS // NW
NBR_GROUP = 4
NBR_NGROUPS = NBR_ROWS_PER_W // NBR_GROUP

EMB_IDX_ROWS = 512
EMB_ROWS_PER_W = EMB_IDX_ROWS // NW
EMB_GROUPS = (4, 4, 4, 4)


def _sc_gather(table, idx2d, rows_per_worker, groups, n_loop_groups):
    n_idx_rows, _ = idx2d.shape
    D = table.shape[1]
    gmax = max(groups) if groups else 0
    g0 = groups[0] if groups else NBR_GROUP
    buf_rows = max(gmax, g0)

    def body(tab, idx, out, idx_v, rows_v, sem):
        c = lax.axis_index("c")
        s = lax.axis_index("s")
        w = s * 2 + c
        base = w * rows_per_worker

        def run_group(r0, g):
            pltpu.sync_copy(idx.at[pl.ds(r0, g)], idx_v.at[pl.ds(0, g)])
            cps = [
                pltpu.async_copy(
                    tab.at[idx_v.at[j]],
                    rows_v.at[pl.ds(j * LANES, LANES)],
                    sem,
                )
                for j in range(g)
            ]
            for cp in cps:
                cp.wait()
            pltpu.sync_copy(
                rows_v.at[pl.ds(0, g * LANES)],
                out.at[pl.ds(r0 * LANES, g * LANES)],
            )

        if n_loop_groups > 0:
            def loop_body(gi, carry):
                run_group(base + gi * g0, g0)
                return carry
            lax.fori_loop(0, n_loop_groups, loop_body, 0)

        off = n_loop_groups * g0
        for g in groups if n_loop_groups == 0 else ():
            run_group(base + off, g)
            off += g

    fn = pl.kernel(
        body,
        out_type=jax.ShapeDtypeStruct((n_idx_rows * LANES, D), table.dtype),
        mesh=plsc.VectorSubcoreMesh(core_axis_name="c", subcore_axis_name="s"),
        scratch_types=[
            pltpu.VMEM((buf_rows, LANES), jnp.int32),
            pltpu.VMEM((buf_rows * LANES, D), table.dtype),
            pltpu.SemaphoreType.DMA,
        ],
    )
    return fn(table, idx2d)


def _sigmoid(x):
    return 1.0 / (1.0 + jnp.exp(-x))


def _softplus(x):
    return jnp.maximum(x, 0.0) + jnp.log1p(jnp.exp(-jnp.abs(x)))


def _conv_body(x_ref, g_ref, e_ref, wsf, wsc, wnf, wnc, wef, wec, bff, bfc,
               xo_ref, s_ref, ss_ref):
    x = x_ref[...]
    sf = jnp.dot(x, wsf[...], preferred_element_type=jnp.float32) + bff[...]
    sc = jnp.dot(x, wsc[...], preferred_element_type=jnp.float32) + bfc[...]
    gm = g_ref[...]
    em = e_ref[...]
    ff = (jnp.dot(gm, wnf[...], preferred_element_type=jnp.float32)
          + jnp.dot(em, wef[...], preferred_element_type=jnp.float32))
    cc = (jnp.dot(gm, wnc[...], preferred_element_type=jnp.float32)
          + jnp.dot(em, wec[...], preferred_element_type=jnp.float32))
    f3 = ff.reshape(B, M, F) + sf[:, None, :]
    c3 = cc.reshape(B, M, F) + sc[:, None, :]
    msg = jnp.sum(_sigmoid(f3) * _softplus(c3), axis=1)
    xn = x[:, :F] + msg
    xo_ref[...] = xn
    s_ref[...] = jnp.broadcast_to(jnp.sum(xn, axis=0, keepdims=True), (8, F))
    ss_ref[...] = jnp.broadcast_to(
        jnp.sum(xn * xn, axis=0, keepdims=True), (8, F))


def _conv_call(x, g, e, wsf, wsc, wnf, wnc, wef, wec, bff, bfc):
    full = lambda s: pl.BlockSpec(s, lambda i: (0, 0))
    return pl.pallas_call(
        _conv_body,
        grid=(NBLK,),
        in_specs=[
            pl.BlockSpec((B, 2 * F), lambda i: (i, 0)),
            pl.BlockSpec((B * M, 2 * F), lambda i: (i, 0)),
            pl.BlockSpec((B * M, NBR), lambda i: (i, 0)),
            full((2 * F, F)), full((2 * F, F)), full((2 * F, F)),
            full((2 * F, F)),
            full((NBR, F)), full((NBR, F)), full((1, F)), full((1, F)),
        ],
        out_specs=[
            pl.BlockSpec((B, F), lambda i: (i, 0)),
            pl.BlockSpec((8, F), lambda i: (i, 0)),
            pl.BlockSpec((8, F), lambda i: (i, 0)),
        ],
        out_shape=[
            jax.ShapeDtypeStruct((N, F), jnp.float32),
            jax.ShapeDtypeStruct((8 * NBLK, F), jnp.float32),
            jax.ShapeDtypeStruct((8 * NBLK, F), jnp.float32),
        ],
    )(x, g, e, wsf, wsc, wnf, wnc, wef, wec, bff, bfc)


def _bn_stats(s_ref, ss_ref):
    mean = jnp.sum(s_ref[...], axis=0, keepdims=True) * (1.0 / (8 * N))
    ex2 = jnp.sum(ss_ref[...], axis=0, keepdims=True) * (1.0 / (8 * N))
    var = ex2 - mean * mean
    rstd = lax.rsqrt(var + EPS)
    return mean, rstd


def _bn_body(x_ref, s_ref, ss_ref, g_ref, b_ref, o_ref):
    mean, rstd = _bn_stats(s_ref, ss_ref)
    xh = (x_ref[...] - mean) * rstd
    y = _softplus(xh * g_ref[...] + b_ref[...])
    o_ref[...] = jnp.concatenate([y, jnp.zeros((B, F), jnp.float32)], axis=1)


def _bn_call(x, s, ss, g, b):
    full = lambda shp: pl.BlockSpec(shp, lambda i: (0, 0))
    return pl.pallas_call(
        _bn_body,
        grid=(NBLK,),
        in_specs=[
            pl.BlockSpec((B, F), lambda i: (i, 0)),
            full((8 * NBLK, F)), full((8 * NBLK, F)),
            full((1, F)), full((1, F)),
        ],
        out_specs=pl.BlockSpec((B, 2 * F), lambda i: (i, 0)),
        out_shape=jax.ShapeDtypeStruct((N, 2 * F), jnp.float32),
    )(x, s, ss, g, b)


def _pool_body(x_ref, s_ref, ss_ref, g_ref, b_ref, cid_ref,
               w1, b1, w2, b2, wo, bo, out_ref, acc, cnt):
    i = pl.program_id(0)

    @pl.when(i == 0)
    def _():
        acc[...] = jnp.zeros_like(acc)
        cnt[...] = jnp.zeros_like(cnt)

    mean, rstd = _bn_stats(s_ref, ss_ref)
    xh = (x_ref[...] - mean) * rstd
    y = _softplus(xh * g_ref[...] + b_ref[...])

    cid = cid_ref[...].reshape(1, B)
    oht = (cid == lax.broadcasted_iota(jnp.int32, (NCRY, B), 0)
           ).astype(jnp.float32)
    acc[...] += jnp.dot(oht, y, preferred_element_type=jnp.float32)
    cnt[...] += jnp.dot(oht, jnp.ones((B, F), jnp.float32),
                        preferred_element_type=jnp.float32)

    @pl.when(i == NBLK - 1)
    def _():
        cf = acc[...] / jnp.maximum(cnt[...], 1.0)
        h1 = _softplus(jnp.dot(cf, w1[...],
                               preferred_element_type=jnp.float32) + b1[...])
        h2 = _softplus(jnp.dot(h1, w2[...],
                               preferred_element_type=jnp.float32) + b2[...])
        out_ref[...] = jnp.dot(h2, wo[...],
                               preferred_element_type=jnp.float32) + bo[...]


def _pool_call(x, s, ss, g, b, cid3, w1, b1, w2, b2, wo, bo):
    full2 = lambda shp: pl.BlockSpec(shp, lambda i: (0, 0))
    return pl.pallas_call(
        _pool_body,
        grid=(NBLK,),
        in_specs=[
            pl.BlockSpec((B, F), lambda i: (i, 0)),
            full2((8 * NBLK, F)), full2((8 * NBLK, F)),
            full2((1, F)), full2((1, F)),
            pl.BlockSpec((1, 1, B), lambda i: (i, 0, 0)),
            full2((F, H)), full2((1, H)), full2((H, F)), full2((1, F)),
            full2((F, 1)), full2((1, 1)),
        ],
        out_specs=pl.BlockSpec((NCRY, 1), lambda i: (0, 0)),
        out_shape=jax.ShapeDtypeStruct((NCRY, 1), jnp.float32),
        scratch_shapes=[
            pltpu.VMEM((NCRY, F), jnp.float32),
            pltpu.VMEM((NCRY, F), jnp.float32),
        ],
    )(x, s, ss, g, b, cid3, w1, b1, w2, b2, wo, bo)


def kernel(atom_z, nbr_fea, nbr_idx, crystal_atom_idx, atom_emb,
           W_full_0, b_full_0, bn_g_0, bn_b_0,
           W_full_1, b_full_1, bn_g_1, bn_b_1,
           W_full_2, b_full_2, bn_g_2, bn_b_2,
           W1, b1, W2, b2, Wo, bo):
    z_pad = jnp.pad(atom_z.astype(jnp.int32),
                    (0, EMB_IDX_ROWS * LANES - N)).reshape(EMB_IDX_ROWS, LANES)
    nidx = jnp.pad(nbr_idx.reshape(-1).astype(jnp.int32),
                   (0, NBR_IDX_ROWS * LANES - E_TOT)).reshape(NBR_IDX_ROWS,
                                                              LANES)
    e2 = nbr_fea.reshape(E_TOT, NBR)
    cid3 = crystal_atom_idx.astype(jnp.int32).reshape(NBLK, 1, B)

    convs = []
    for (Wf, bf, g, b) in ((W_full_0, b_full_0, bn_g_0, bn_b_0),
                           (W_full_1, b_full_1, bn_g_1, bn_b_1),
                           (W_full_2, b_full_2, bn_g_2, bn_b_2)):
        pad_k = lambda w: jnp.pad(w, ((0, F), (0, 0)))
        convs.append(dict(
            wsf=pad_k(Wf[:F, :F]), wsc=pad_k(Wf[:F, F:]),
            wnf=pad_k(Wf[F:2 * F, :F]), wnc=pad_k(Wf[F:2 * F, F:]),
            wef=Wf[2 * F:, :F], wec=Wf[2 * F:, F:],
            bff=bf[:F].reshape(1, F), bfc=bf[F:].reshape(1, F),
            g=g.reshape(1, F), b=b.reshape(1, F),
        ))

    emb_pad = jnp.pad(atom_emb, ((0, 128 - atom_emb.shape[0]), (0, F)))
    x = _sc_gather(emb_pad, z_pad, EMB_ROWS_PER_W, EMB_GROUPS, 0)

    xp = s = ss = None
    for li in range(3):
        cv = convs[li]
        gath = _sc_gather(x, nidx, NBR_ROWS_PER_W, (NBR_GROUP,), NBR_NGROUPS)
        xp, s, ss = _conv_call(x, gath, e2, cv["wsf"], cv["wsc"], cv["wnf"],
                               cv["wnc"], cv["wef"], cv["wec"], cv["bff"],
                               cv["bfc"])
        if li < 2:
            x = _bn_call(xp, s, ss, cv["g"], cv["b"])

    cv = convs[2]
    return _pool_call(xp, s, ss, cv["g"], cv["b"], cid3,
                      W1, b1.reshape(1, H), W2, b2.reshape(1, F),
                      Wo, bo.reshape(1, 1))


# speedup vs baseline: 1.2965x; 1.2033x over previous
import functools

import jax
import jax.numpy as jnp
from jax import lax
from jax.experimental import pallas as pl
from jax.experimental.pallas import tpu as pltpu
from jax.experimental.pallas import tpu_sc as plsc

N = 50000
M = 16
F = 64
NBR = 16
NCRY = 256
H = 128
EPS = 1e-5

B = 400
NBLK = N // B
E_TOT = N * M

LANES = 128
NW = 32

NBR_IDX_ROWS = 6400
NBR_ROWS_PER_W = NBR_IDX_ROWS // NW

EMB_IDX_ROWS = 512
EMB_ROWS_PER_W = EMB_IDX_ROWS // NW


def _sc_gather(table, idx2d, rows_per_worker, bank_rows):
    n_idx_rows, _ = idx2d.shape
    D = table.shape[1]
    R = bank_rows
    nb = rows_per_worker // R
    assert nb % 2 == 0 and nb >= 2

    def body(tab, idx, out, idx_all, rows_v, sem_g, sem_s):
        c = lax.axis_index("c")
        s = lax.axis_index("s")
        w = s * 2 + c
        base = w * rows_per_worker
        pltpu.sync_copy(idx.at[pl.ds(base, rows_per_worker)], idx_all)

        def fire_gather(bank, j0):
            for b in range(R):
                pltpu.async_copy(
                    tab.at[idx_all.at[j0 + b]],
                    rows_v.at[pl.ds((bank * R + b) * LANES, LANES)],
                    sem_g,
                )

        def fire_store(bank, j0):
            for b in range(R):
                pltpu.async_copy(
                    rows_v.at[pl.ds((bank * R + b) * LANES, LANES)],
                    out.at[pl.ds((base + j0 + b) * LANES, LANES)],
                    sem_s,
                )

        def wait_g(n):
            for _ in range(n):
                pltpu.make_async_copy(
                    tab.at[idx_all.at[0]],
                    rows_v.at[pl.ds(0, LANES)], sem_g).wait()

        def wait_s(n):
            for _ in range(n):
                pltpu.make_async_copy(
                    rows_v.at[pl.ds(0, LANES)],
                    out.at[pl.ds(base * LANES, LANES)], sem_s).wait()

        fire_gather(0, 0)

        def loop_body(k2, carry):
            a = 2 * k2
            wait_g(R)

            @pl.when(k2 > 0)
            def _():
                wait_s(R)
            fire_gather(1, (a + 1) * R)
            fire_store(0, a * R)
            wait_g(R)
            wait_s(R)

            @pl.when(k2 < nb // 2 - 1)
            def _():
                fire_gather(0, (a + 2) * R)
            fire_store(1, (a + 1) * R)
            return carry

        lax.fori_loop(0, nb // 2, loop_body, 0)
        wait_s(R)

    fn = pl.kernel(
        body,
        out_type=jax.ShapeDtypeStruct((n_idx_rows * LANES, D), table.dtype),
        mesh=plsc.VectorSubcoreMesh(core_axis_name="c", subcore_axis_name="s"),
        scratch_types=[
            pltpu.VMEM((rows_per_worker, LANES), jnp.int32),
            pltpu.VMEM((2 * R * LANES, D), table.dtype),
            pltpu.SemaphoreType.DMA,
            pltpu.SemaphoreType.DMA,
        ],
    )
    return fn(table, idx2d)


def _sigmoid(x):
    return 1.0 / (1.0 + jnp.exp(-x))


def _softplus(x):
    return jnp.maximum(x, 0.0) + jnp.log1p(jnp.exp(-jnp.abs(x)))


def _conv_body(x_ref, g_ref, e_ref, ws, wn, we, bf, xo_ref, s_ref, ss_ref):
    x = x_ref[...]
    sfc = jnp.dot(x, ws[...], preferred_element_type=jnp.float32) + bf[...]
    gm = g_ref[...]
    em = e_ref[...]
    h = (jnp.dot(gm, wn[...], preferred_element_type=jnp.float32)
         + jnp.dot(em, we[...], preferred_element_type=jnp.float32))
    h3 = h.reshape(B, M, 2 * F) + sfc[:, None, :]
    msg = jnp.sum(_sigmoid(h3[..., :F]) * _softplus(h3[..., F:]), axis=1)
    xn = x[:, :F] + msg
    xo_ref[...] = xn
    s_ref[...] = jnp.broadcast_to(jnp.sum(xn, axis=0, keepdims=True), (8, F))
    ss_ref[...] = jnp.broadcast_to(
        jnp.sum(xn * xn, axis=0, keepdims=True), (8, F))


def _conv_call(x, g, e, ws, wn, we, bf):
    full = lambda s: pl.BlockSpec(s, lambda i: (0, 0))
    return pl.pallas_call(
        _conv_body,
        grid=(NBLK,),
        in_specs=[
            pl.BlockSpec((B, 2 * F), lambda i: (i, 0)),
            pl.BlockSpec((B * M, 2 * F), lambda i: (i, 0)),
            pl.BlockSpec((B * M, NBR), lambda i: (i, 0)),
            full((2 * F, 2 * F)), full((2 * F, 2 * F)),
            full((NBR, 2 * F)), full((1, 2 * F)),
        ],
        out_specs=[
            pl.BlockSpec((B, F), lambda i: (i, 0)),
            pl.BlockSpec((8, F), lambda i: (i, 0)),
            pl.BlockSpec((8, F), lambda i: (i, 0)),
        ],
        out_shape=[
            jax.ShapeDtypeStruct((N, F), jnp.float32),
            jax.ShapeDtypeStruct((8 * NBLK, F), jnp.float32),
            jax.ShapeDtypeStruct((8 * NBLK, F), jnp.float32),
        ],
    )(x, g, e, ws, wn, we, bf)


def _bn_stats(s_ref, ss_ref):
    mean = jnp.sum(s_ref[...], axis=0, keepdims=True) * (1.0 / (8 * N))
    ex2 = jnp.sum(ss_ref[...], axis=0, keepdims=True) * (1.0 / (8 * N))
    var = ex2 - mean * mean
    rstd = lax.rsqrt(var + EPS)
    return mean, rstd


def _bn_body(x_ref, s_ref, ss_ref, g_ref, b_ref, o_ref):
    mean, rstd = _bn_stats(s_ref, ss_ref)
    xh = (x_ref[...] - mean) * rstd
    y = _softplus(xh * g_ref[...] + b_ref[...])
    o_ref[...] = jnp.concatenate([y, jnp.zeros((B, F), jnp.float32)], axis=1)


def _bn_call(x, s, ss, g, b):
    full = lambda shp: pl.BlockSpec(shp, lambda i: (0, 0))
    return pl.pallas_call(
        _bn_body,
        grid=(NBLK,),
        in_specs=[
            pl.BlockSpec((B, F), lambda i: (i, 0)),
            full((8 * NBLK, F)), full((8 * NBLK, F)),
            full((1, F)), full((1, F)),
        ],
        out_specs=pl.BlockSpec((B, 2 * F), lambda i: (i, 0)),
        out_shape=jax.ShapeDtypeStruct((N, 2 * F), jnp.float32),
    )(x, s, ss, g, b)


def _pool_body(x_ref, s_ref, ss_ref, g_ref, b_ref, cid_ref,
               w1, b1, w2, b2, wo, bo, out_ref, acc, cnt):
    i = pl.program_id(0)

    @pl.when(i == 0)
    def _():
        acc[...] = jnp.zeros_like(acc)
        cnt[...] = jnp.zeros_like(cnt)

    mean, rstd = _bn_stats(s_ref, ss_ref)
    xh = (x_ref[...] - mean) * rstd
    y = _softplus(xh * g_ref[...] + b_ref[...])

    cid = cid_ref[...].reshape(1, B)
    oht = (cid == lax.broadcasted_iota(jnp.int32, (NCRY, B), 0)
           ).astype(jnp.float32)
    acc[...] += jnp.dot(oht, y, preferred_element_type=jnp.float32)
    cnt[...] += jnp.dot(oht, jnp.ones((B, F), jnp.float32),
                        preferred_element_type=jnp.float32)

    @pl.when(i == NBLK - 1)
    def _():
        cf = acc[...] / jnp.maximum(cnt[...], 1.0)
        h1 = _softplus(jnp.dot(cf, w1[...],
                               preferred_element_type=jnp.float32) + b1[...])
        h2 = _softplus(jnp.dot(h1, w2[...],
                               preferred_element_type=jnp.float32) + b2[...])
        out_ref[...] = jnp.dot(h2, wo[...],
                               preferred_element_type=jnp.float32) + bo[...]


def _pool_call(x, s, ss, g, b, cid3, w1, b1, w2, b2, wo, bo):
    full2 = lambda shp: pl.BlockSpec(shp, lambda i: (0, 0))
    return pl.pallas_call(
        _pool_body,
        grid=(NBLK,),
        in_specs=[
            pl.BlockSpec((B, F), lambda i: (i, 0)),
            full2((8 * NBLK, F)), full2((8 * NBLK, F)),
            full2((1, F)), full2((1, F)),
            pl.BlockSpec((1, 1, B), lambda i: (i, 0, 0)),
            full2((F, H)), full2((1, H)), full2((H, F)), full2((1, F)),
            full2((F, 1)), full2((1, 1)),
        ],
        out_specs=pl.BlockSpec((NCRY, 1), lambda i: (0, 0)),
        out_shape=jax.ShapeDtypeStruct((NCRY, 1), jnp.float32),
        scratch_shapes=[
            pltpu.VMEM((NCRY, F), jnp.float32),
            pltpu.VMEM((NCRY, F), jnp.float32),
        ],
    )(x, s, ss, g, b, cid3, w1, b1, w2, b2, wo, bo)


def kernel(atom_z, nbr_fea, nbr_idx, crystal_atom_idx, atom_emb,
           W_full_0, b_full_0, bn_g_0, bn_b_0,
           W_full_1, b_full_1, bn_g_1, bn_b_1,
           W_full_2, b_full_2, bn_g_2, bn_b_2,
           W1, b1, W2, b2, Wo, bo):
    z_pad = jnp.pad(atom_z.astype(jnp.int32),
                    (0, EMB_IDX_ROWS * LANES - N)).reshape(EMB_IDX_ROWS, LANES)
    nidx = jnp.pad(nbr_idx.reshape(-1).astype(jnp.int32),
                   (0, NBR_IDX_ROWS * LANES - E_TOT)).reshape(NBR_IDX_ROWS,
                                                              LANES)
    e2 = nbr_fea.reshape(E_TOT, NBR)
    cid3 = crystal_atom_idx.astype(jnp.int32).reshape(NBLK, 1, B)

    convs = []
    for (Wf, bf, g, b) in ((W_full_0, b_full_0, bn_g_0, bn_b_0),
                           (W_full_1, b_full_1, bn_g_1, bn_b_1),
                           (W_full_2, b_full_2, bn_g_2, bn_b_2)):
        pad_k = lambda w: jnp.pad(w, ((0, F), (0, 0)))
        convs.append(dict(
            ws=pad_k(Wf[:F, :]), wn=pad_k(Wf[F:2 * F, :]),
            we=Wf[2 * F:, :], bf=bf.reshape(1, 2 * F),
            g=g.reshape(1, F), b=b.reshape(1, F),
        ))

    emb_pad = jnp.pad(atom_emb, ((0, 128 - atom_emb.shape[0]), (0, F)))
    x = _sc_gather(emb_pad, z_pad, EMB_ROWS_PER_W, 2)

    xp = s = ss = None
    for li in range(3):
        cv = convs[li]
        gath = _sc_gather(x, nidx, NBR_ROWS_PER_W, 2)
        xp, s, ss = _conv_call(x, gath, e2, cv["ws"], cv["wn"], cv["we"],
                               cv["bf"])
        if li < 2:
            x = _bn_call(xp, s, ss, cv["g"], cv["b"])

    cv = convs[2]
    return _pool_call(xp, s, ss, cv["g"], cv["b"], cid3,
                      W1, b1.reshape(1, H), W2, b2.reshape(1, F),
                      Wo, bo.reshape(1, 1))


# speedup vs baseline: 1.4921x; 1.1508x over previous
import functools

import jax
import jax.numpy as jnp
from jax import lax
from jax.experimental import pallas as pl
from jax.experimental.pallas import tpu as pltpu
from jax.experimental.pallas import tpu_sc as plsc

N = 50000
M = 16
F = 64
NBR = 16
NCRY = 256
H = 128
EPS = 1e-5

B = 400
NBLK = N // B
E_TOT = N * M

LANES = 128
NW = 32

NBR_IDX_ROWS = 6400
NBR_ROWS_PER_W = NBR_IDX_ROWS // NW

NBR_BANK_ROWS = 2


def _sc_gather(table, idx2d, rows_per_worker, bank_rows):
    n_idx_rows, _ = idx2d.shape
    D = table.shape[1]
    R = bank_rows
    nb = rows_per_worker // R
    assert nb % 2 == 0 and nb >= 2

    def body(tab, idx, out, idx_all, rows_v, sem_g, sem_s):
        c = lax.axis_index("c")
        s = lax.axis_index("s")
        w = s * 2 + c
        base = w * rows_per_worker
        pltpu.sync_copy(idx.at[pl.ds(base, rows_per_worker)], idx_all)

        def fire_gather(bank, j0):
            for b in range(R):
                pltpu.async_copy(
                    tab.at[idx_all.at[j0 + b]],
                    rows_v.at[pl.ds((bank * R + b) * LANES, LANES)],
                    sem_g,
                )

        def fire_store(bank, j0):
            for b in range(R):
                pltpu.async_copy(
                    rows_v.at[pl.ds((bank * R + b) * LANES, LANES)],
                    out.at[pl.ds((base + j0 + b) * LANES, LANES)],
                    sem_s,
                )

        def wait_g(n):
            for _ in range(n):
                pltpu.make_async_copy(
                    tab.at[idx_all.at[0]],
                    rows_v.at[pl.ds(0, LANES)], sem_g).wait()

        def wait_s(n):
            for _ in range(n):
                pltpu.make_async_copy(
                    rows_v.at[pl.ds(0, LANES)],
                    out.at[pl.ds(base * LANES, LANES)], sem_s).wait()

        fire_gather(0, 0)

        def loop_body(k2, carry):
            a = 2 * k2
            wait_g(R)

            @pl.when(k2 > 0)
            def _():
                wait_s(R)
            fire_gather(1, (a + 1) * R)
            fire_store(0, a * R)
            wait_g(R)
            wait_s(R)

            @pl.when(k2 < nb // 2 - 1)
            def _():
                fire_gather(0, (a + 2) * R)
            fire_store(1, (a + 1) * R)
            return carry

        lax.fori_loop(0, nb // 2, loop_body, 0)
        wait_s(R)

    fn = pl.kernel(
        body,
        out_type=jax.ShapeDtypeStruct((n_idx_rows * LANES, D), table.dtype),
        mesh=plsc.VectorSubcoreMesh(core_axis_name="c", subcore_axis_name="s"),
        scratch_types=[
            pltpu.VMEM((rows_per_worker, LANES), jnp.int32),
            pltpu.VMEM((2 * R * LANES, D), table.dtype),
            pltpu.SemaphoreType.DMA,
            pltpu.SemaphoreType.DMA,
        ],
    )
    return fn(table, idx2d)


def _sigmoid(x):
    return 1.0 / (1.0 + jnp.exp(-x))


def _softplus(x):
    return jnp.maximum(x, 0.0) + jnp.log1p(jnp.exp(-jnp.abs(x)))


def _conv_body(x_ref, g_ref, e_ref, ws, we, bf, xo_ref, s_ref, ss_ref):
    x = x_ref[...]
    sfc = jnp.dot(x, ws[...], preferred_element_type=jnp.float32) + bf[...]
    gm = g_ref[...]
    em = e_ref[...]
    h = gm + jnp.dot(em, we[...], preferred_element_type=jnp.float32)
    h3 = h.reshape(B, M, 2 * F) + sfc[:, None, :]
    msg = jnp.sum(_sigmoid(h3[..., :F]) * _softplus(h3[..., F:]), axis=1)
    xn = x + msg
    xo_ref[...] = xn
    s_ref[...] = jnp.broadcast_to(jnp.sum(xn, axis=0, keepdims=True), (8, F))
    ss_ref[...] = jnp.broadcast_to(
        jnp.sum(xn * xn, axis=0, keepdims=True), (8, F))


def _conv_call(x, g, e, ws, we, bf):
    full = lambda s: pl.BlockSpec(s, lambda i: (0, 0))
    return pl.pallas_call(
        _conv_body,
        grid=(NBLK,),
        in_specs=[
            pl.BlockSpec((B, F), lambda i: (i, 0)),
            pl.BlockSpec((B * M, 2 * F), lambda i: (i, 0)),
            pl.BlockSpec((B * M, NBR), lambda i: (i, 0)),
            full((F, 2 * F)),
            full((NBR, 2 * F)), full((1, 2 * F)),
        ],
        out_specs=[
            pl.BlockSpec((B, F), lambda i: (i, 0)),
            pl.BlockSpec((8, F), lambda i: (i, 0)),
            pl.BlockSpec((8, F), lambda i: (i, 0)),
        ],
        out_shape=[
            jax.ShapeDtypeStruct((N, F), jnp.float32),
            jax.ShapeDtypeStruct((8 * NBLK, F), jnp.float32),
            jax.ShapeDtypeStruct((8 * NBLK, F), jnp.float32),
        ],
    )(x, g, e, ws, we, bf)


def _bn_stats(s_ref, ss_ref):
    mean = jnp.sum(s_ref[...], axis=0, keepdims=True) * (1.0 / (8 * N))
    ex2 = jnp.sum(ss_ref[...], axis=0, keepdims=True) * (1.0 / (8 * N))
    var = ex2 - mean * mean
    rstd = lax.rsqrt(var + EPS)
    return mean, rstd


def _bn_body(x_ref, s_ref, ss_ref, g_ref, b_ref, wn_ref, o_ref, t_ref):
    mean, rstd = _bn_stats(s_ref, ss_ref)
    xh = (x_ref[...] - mean) * rstd
    y = _softplus(xh * g_ref[...] + b_ref[...])
    o_ref[...] = y
    t_ref[...] = jnp.dot(y, wn_ref[...], preferred_element_type=jnp.float32)


def _bn_call(x, s, ss, g, b, wn_next):
    full = lambda shp: pl.BlockSpec(shp, lambda i: (0, 0))
    return pl.pallas_call(
        _bn_body,
        grid=(NBLK,),
        in_specs=[
            pl.BlockSpec((B, F), lambda i: (i, 0)),
            full((8 * NBLK, F)), full((8 * NBLK, F)),
            full((1, F)), full((1, F)), full((F, 2 * F)),
        ],
        out_specs=[
            pl.BlockSpec((B, F), lambda i: (i, 0)),
            pl.BlockSpec((B, 2 * F), lambda i: (i, 0)),
        ],
        out_shape=[
            jax.ShapeDtypeStruct((N, F), jnp.float32),
            jax.ShapeDtypeStruct((N, 2 * F), jnp.float32),
        ],
    )(x, s, ss, g, b, wn_next)


def _embed_body(z_ref, emb_ref, wn_ref, xo_ref, t_ref):
    z = z_ref[...].reshape(1, B)
    oht = (z == lax.broadcasted_iota(jnp.int32, (128, B), 0)
           ).astype(jnp.float32)
    x128 = lax.dot_general(oht, emb_ref[...], (((0,), (0,)), ((), ())),
                           preferred_element_type=jnp.float32)
    x0 = x128[:, :F]
    xo_ref[...] = x0
    t_ref[...] = jnp.dot(x0, wn_ref[...], preferred_element_type=jnp.float32)


def _embed_call(z3, emb_pad, wn0):
    return pl.pallas_call(
        _embed_body,
        grid=(NBLK,),
        in_specs=[
            pl.BlockSpec((1, 1, B), lambda i: (i, 0, 0)),
            pl.BlockSpec((128, 2 * F), lambda i: (0, 0)),
            pl.BlockSpec((F, 2 * F), lambda i: (0, 0)),
        ],
        out_specs=[
            pl.BlockSpec((B, F), lambda i: (i, 0)),
            pl.BlockSpec((B, 2 * F), lambda i: (i, 0)),
        ],
        out_shape=[
            jax.ShapeDtypeStruct((N, F), jnp.float32),
            jax.ShapeDtypeStruct((N, 2 * F), jnp.float32),
        ],
    )(z3, emb_pad, wn0)


def _pool_body(x_ref, s_ref, ss_ref, g_ref, b_ref, cid_ref,
               w1, b1, w2, b2, wo, bo, out_ref, acc, cnt):
    i = pl.program_id(0)

    @pl.when(i == 0)
    def _():
        acc[...] = jnp.zeros_like(acc)
        cnt[...] = jnp.zeros_like(cnt)

    mean, rstd = _bn_stats(s_ref, ss_ref)
    xh = (x_ref[...] - mean) * rstd
    y = _softplus(xh * g_ref[...] + b_ref[...])

    cid = cid_ref[...].reshape(1, B)
    oht = (cid == lax.broadcasted_iota(jnp.int32, (NCRY, B), 0)
           ).astype(jnp.float32)
    acc[...] += jnp.dot(oht, y, preferred_element_type=jnp.float32)
    cnt[...] += jnp.dot(oht, jnp.ones((B, F), jnp.float32),
                        preferred_element_type=jnp.float32)

    @pl.when(i == NBLK - 1)
    def _():
        cf = acc[...] / jnp.maximum(cnt[...], 1.0)
        h1 = _softplus(jnp.dot(cf, w1[...],
                               preferred_element_type=jnp.float32) + b1[...])
        h2 = _softplus(jnp.dot(h1, w2[...],
                               preferred_element_type=jnp.float32) + b2[...])
        out_ref[...] = jnp.dot(h2, wo[...],
                               preferred_element_type=jnp.float32) + bo[...]


def _pool_call(x, s, ss, g, b, cid3, w1, b1, w2, b2, wo, bo):
    full2 = lambda shp: pl.BlockSpec(shp, lambda i: (0, 0))
    return pl.pallas_call(
        _pool_body,
        grid=(NBLK,),
        in_specs=[
            pl.BlockSpec((B, F), lambda i: (i, 0)),
            full2((8 * NBLK, F)), full2((8 * NBLK, F)),
            full2((1, F)), full2((1, F)),
            pl.BlockSpec((1, 1, B), lambda i: (i, 0, 0)),
            full2((F, H)), full2((1, H)), full2((H, F)), full2((1, F)),
            full2((F, 1)), full2((1, 1)),
        ],
        out_specs=pl.BlockSpec((NCRY, 1), lambda i: (0, 0)),
        out_shape=jax.ShapeDtypeStruct((NCRY, 1), jnp.float32),
        scratch_shapes=[
            pltpu.VMEM((NCRY, F), jnp.float32),
            pltpu.VMEM((NCRY, F), jnp.float32),
        ],
    )(x, s, ss, g, b, cid3, w1, b1, w2, b2, wo, bo)


def kernel(atom_z, nbr_fea, nbr_idx, crystal_atom_idx, atom_emb,
           W_full_0, b_full_0, bn_g_0, bn_b_0,
           W_full_1, b_full_1, bn_g_1, bn_b_1,
           W_full_2, b_full_2, bn_g_2, bn_b_2,
           W1, b1, W2, b2, Wo, bo):
    z3 = atom_z.astype(jnp.int32).reshape(NBLK, 1, B)
    nidx = jnp.pad(nbr_idx.reshape(-1).astype(jnp.int32),
                   (0, NBR_IDX_ROWS * LANES - E_TOT)).reshape(NBR_IDX_ROWS,
                                                              LANES)
    e2 = nbr_fea.reshape(E_TOT, NBR).astype(jnp.bfloat16)
    cid3 = crystal_atom_idx.astype(jnp.int32).reshape(NBLK, 1, B)

    convs = []
    for (Wf, bf, g, b) in ((W_full_0, b_full_0, bn_g_0, bn_b_0),
                           (W_full_1, b_full_1, bn_g_1, bn_b_1),
                           (W_full_2, b_full_2, bn_g_2, bn_b_2)):
        convs.append(dict(
            ws=Wf[:F, :], wn=Wf[F:2 * F, :],
            we=Wf[2 * F:, :].astype(jnp.bfloat16), bf=bf.reshape(1, 2 * F),
            g=g.reshape(1, F), b=b.reshape(1, F),
        ))

    emb_pad = jnp.pad(atom_emb, ((0, 128 - atom_emb.shape[0]), (0, F)))
    x, xt = _embed_call(z3, emb_pad, convs[0]["wn"])

    xp = s = ss = None
    for li in range(3):
        cv = convs[li]
        gath = _sc_gather(xt, nidx, NBR_ROWS_PER_W, NBR_BANK_ROWS)
        xp, s, ss = _conv_call(x, gath, e2, cv["ws"], cv["we"], cv["bf"])
        if li < 2:
            x, xt = _bn_call(xp, s, ss, cv["g"], cv["b"], convs[li + 1]["wn"])

    cv = convs[2]
    return _pool_call(xp, s, ss, cv["g"], cv["b"], cid3,
                      W1, b1.reshape(1, H), W2, b2.reshape(1, F),
                      Wo, bo.reshape(1, 1))
